# baseline (speedup 1.0000x reference)
import functools

import jax
import jax.numpy as jnp
from jax import lax
from jax.experimental import pallas as pl
from jax.experimental.pallas import tpu as pltpu
from jax.experimental.pallas import tpu_sc as plsc

B = 16384
F = 26
V = 100000
D = 32
ND = 13
H = 64

NC = 2
NS = 16
NW = NC * NS

BPW = B // NW
GL = 64
BG = BPW // GL

VP = 100096
NVC = V // 128

_BN_INV = 1.0 / (1.0 + 1e-5) ** 0.5


def _pack_chunk(yc):
    pieces = [yc[32 * m : 32 * (m + 1), :] for m in range(4)]
    return jnp.concatenate(pieces, axis=1).reshape(4096)


def _transpose_body(in_ref, out_ref):
    def chunk(k, carry):
        xc = in_ref[0, :, pl.ds(k * 128, 128)]
        out_ref[pl.ds(k * 4096, 4096)] = _pack_chunk(jnp.swapaxes(xc, 0, 1))
        return carry

    jax.lax.fori_loop(0, NVC, chunk, 0)
    xt = in_ref[0, :, pl.ds(NVC * 128, 32)]
    yt = jnp.swapaxes(xt, 0, 1)
    out_ref[pl.ds(NVC * 4096, 4096)] = jnp.concatenate([yt] * 4, axis=1).reshape(4096)


def _tc_transpose(t_view):
    return pl.pallas_call(
        _transpose_body,
        grid=(F,),
        in_specs=[pl.BlockSpec((1, D, V), lambda f: (f, 0, 0))],
        out_specs=pl.BlockSpec((VP * D,), lambda f: (f,)),
        out_shape=jax.ShapeDtypeStruct((F * VP * D,), jnp.float32),
    )(t_view)


def _sc_gather(table2d, idx4d):
    mesh = plsc.VectorSubcoreMesh(core_axis_name="c", subcore_axis_name="s")

    @functools.partial(
        pl.kernel,
        out_type=jax.ShapeDtypeStruct((B, F * D), jnp.float32),
        mesh=mesh,
        scratch_types=[
            pltpu.VMEM((BG, F, GL), jnp.int32),
            pltpu.VMEM((2, F, GL, D), jnp.float32),
            pltpu.SemaphoreType.DMA,
            pltpu.SemaphoreType.DMA,
        ],
        compiler_params=pltpu.CompilerParams(use_tc_tiling_on_sc=False),
    )
    def gather_kernel(table_hbm, idx_hbm, out_hbm, idx_v, buf_v, gsem, wsem):
        wid = lax.axis_index("s") * NC + lax.axis_index("c")
        base = wid * BPW
        pltpu.sync_copy(idx_hbm.at[wid], idx_v)

        def fire_gathers(g, p):
            @pl.loop(0, F)
            def _f(f):
                pltpu.async_copy(
                    table_hbm.at[idx_v.at[g, f]],
                    buf_v.at[p, f],
                    gsem,
                )

        def drain_gathers(p):
            @pl.loop(0, F)
            def _f(f):
                pltpu.make_async_copy(
                    table_hbm.at[idx_v.at[0, 0]], buf_v.at[p, 0], gsem
                ).wait()

        def fire_writes(g, p):
            b0 = base + g * GL

            @pl.loop(0, F)
            def _f(f):
                pltpu.async_copy(
                    buf_v.at[p, f],
                    out_hbm.at[pl.ds(b0, GL), pl.ds(f * D, D)],
                    wsem,
                )

        def drain_writes(p):
            @pl.loop(0, F)
            def _f(f):
                pltpu.make_async_copy(
                    buf_v.at[p, 0],
                    out_hbm.at[pl.ds(base, GL), pl.ds(0, D)],
                    wsem,
                ).wait()

        fire_gathers(0, 0)
        drain_gathers(0)
        fire_gathers(1, 1)
        fire_writes(0, 0)

        @pl.loop(2, BG, step=2)
        def _g(g):
            drain_gathers(1)
            drain_writes(0)
            fire_gathers(g, 0)
            fire_writes(g - 1, 1)
            drain_gathers(0)
            drain_writes(1)
            fire_gathers(g + 1, 1)
            fire_writes(g, 0)

        drain_gathers(1)
        drain_writes(0)
        fire_writes(BG - 1, 1)
        drain_writes(1)

    return gather_kernel(table2d, idx4d)


def _mlp_body(emb_ref, num_ref, w1a_ref, w1b_ref, b1_ref, w2_ref, b2_ref,
              bnw_ref, bnb_ref, out_ref):
    x = emb_ref[...]
    h = jnp.dot(x, w1a_ref[...], preferred_element_type=jnp.float32)
    num_n = num_ref[...] * (bnw_ref[...] * _BN_INV) + bnb_ref[...]
    h = h + jnp.dot(num_n, w1b_ref[...], preferred_element_type=jnp.float32)
    h = jnp.maximum(h + b1_ref[...], 0.0)
    out_ref[...] = jnp.dot(h, w2_ref[...], preferred_element_type=jnp.float32) + b2_ref[...]


def _mlp(emb_concat, num, W1, b1, W2, b2, bn_w, bn_b):
    BM = 1024
    W1a = W1[: F * D]
    W1b = W1[F * D :]
    return pl.pallas_call(
        _mlp_body,
        grid=(B // BM,),
        in_specs=[
            pl.BlockSpec((BM, F * D), lambda i: (i, 0)),
            pl.BlockSpec((BM, ND), lambda i: (i, 0)),
            pl.BlockSpec((F * D, H), lambda i: (0, 0)),
            pl.BlockSpec((ND, H), lambda i: (0, 0)),
            pl.BlockSpec((1, H), lambda i: (0, 0)),
            pl.BlockSpec((H, 1), lambda i: (0, 0)),
            pl.BlockSpec((1, 1), lambda i: (0, 0)),
            pl.BlockSpec((1, ND), lambda i: (0, 0)),
            pl.BlockSpec((1, ND), lambda i: (0, 0)),
        ],
        out_specs=pl.BlockSpec((BM, 1), lambda i: (i, 0)),
        out_shape=jax.ShapeDtypeStruct((B, 1), jnp.float32),
    )(emb_concat, num, W1a, W1b, b1.reshape(1, H), W2, b2.reshape(1, 1),
      bn_w.reshape(1, ND), bn_b.reshape(1, ND))


def kernel(cat_idx, num, emb_tables, W1, b1, W2, b2, bn_w, bn_b):
    t_view = jnp.transpose(emb_tables, (0, 2, 1))
    table2d = _tc_transpose(t_view).reshape(F * VP, D)
    v = cat_idx.astype(jnp.int32)
    perm_idx = (v // 128) * 128 + (v % 32) * 4 + (v % 128) // 32
    flat_idx = perm_idx + (jnp.arange(F, dtype=jnp.int32) * VP)[None, :]
    idx4d = flat_idx.reshape(NW, BG, GL, F).transpose(0, 1, 3, 2)
    emb_concat = _sc_gather(table2d, idx4d)
    out = _mlp(emb_concat, num, W1, b1, W2, b2, bn_w, bn_b)
    return (out, emb_concat)


# speedup vs baseline: 3.8326x; 3.8326x over previous
import functools

import jax
import jax.numpy as jnp
from jax import lax
from jax.experimental import pallas as pl
from jax.experimental.pallas import tpu as pltpu
from jax.experimental.pallas import tpu_sc as plsc

B = 16384
F = 26
V = 100000
D = 32
ND = 13
H = 64

NC = 2
NS = 16
NW = NC * NS

BPW = B // NW
GL = 64
BG = BPW // GL

SCW = 1024
NSC = V // SCW
TW = V - NSC * SCW
VP = (NSC + 1) * SCW

_BN_INV = 1.0 / (1.0 + 1e-5) ** 0.5


def _pack_pieces(yc, n):
    pieces = [yc[32 * m : 32 * (m + 1), :] for m in range(n)]
    pieces += [yc[:32, :]] * (32 - n)
    return jnp.concatenate(pieces, axis=1).reshape(32 * SCW)


def _transpose_body(in_ref, out_ref):
    def chunk(k, carry):
        xc = in_ref[0, :, pl.ds(k * SCW, SCW)]
        yc = jnp.swapaxes(xc, 0, 1)
        out_ref[pl.ds(k * 32 * SCW, 32 * SCW)] = _pack_pieces(yc, 32)
        return carry

    jax.lax.fori_loop(0, NSC, chunk, 0, unroll=2)
    xt = in_ref[0, :, pl.ds(NSC * SCW, TW)]
    yt = jnp.swapaxes(xt, 0, 1)
    out_ref[pl.ds(NSC * 32 * SCW, 32 * SCW)] = _pack_pieces(yt, TW // 32)


def _tc_transpose(t_view):
    return pl.pallas_call(
        _transpose_body,
        grid=(F,),
        in_specs=[pl.BlockSpec((1, D, V), lambda f: (f, 0, 0))],
        out_specs=pl.BlockSpec((VP * D,), lambda f: (f,)),
        out_shape=jax.ShapeDtypeStruct((F * VP * D,), jnp.float32),
    )(t_view)


def _sc_gather(table2d, idx4d):
    mesh = plsc.VectorSubcoreMesh(core_axis_name="c", subcore_axis_name="s")

    @functools.partial(
        pl.kernel,
        out_type=jax.ShapeDtypeStruct((B, F * D), jnp.float32),
        mesh=mesh,
        scratch_types=[
            pltpu.VMEM((BG, F, GL), jnp.int32),
            pltpu.VMEM((2, F, GL, D), jnp.float32),
            pltpu.SemaphoreType.DMA,
            pltpu.SemaphoreType.DMA,
        ],
        compiler_params=pltpu.CompilerParams(use_tc_tiling_on_sc=False),
    )
    def gather_kernel(table_hbm, idx_hbm, out_hbm, idx_v, buf_v, gsem, wsem):
        wid = lax.axis_index("s") * NC + lax.axis_index("c")
        base = wid * BPW
        pltpu.sync_copy(idx_hbm.at[wid], idx_v)

        def fire_gathers(g, p):
            @pl.loop(0, F)
            def _f(f):
                pltpu.async_copy(
                    table_hbm.at[idx_v.at[g, f]],
                    buf_v.at[p, f],
                    gsem,
                )

        def drain_gathers(p):
            @pl.loop(0, F)
            def _f(f):
                pltpu.make_async_copy(
                    table_hbm.at[idx_v.at[0, 0]], buf_v.at[p, 0], gsem
                ).wait()

        def fire_writes(g, p):
            b0 = base + g * GL

            @pl.loop(0, F)
            def _f(f):
                pltpu.async_copy(
                    buf_v.at[p, f],
                    out_hbm.at[pl.ds(b0, GL), pl.ds(f * D, D)],
                    wsem,
                )

        def drain_writes(p):
            @pl.loop(0, F)
            def _f(f):
                pltpu.make_async_copy(
                    buf_v.at[p, 0],
                    out_hbm.at[pl.ds(base, GL), pl.ds(0, D)],
                    wsem,
                ).wait()

        fire_gathers(0, 0)
        drain_gathers(0)
        fire_gathers(1, 1)
        fire_writes(0, 0)

        @pl.loop(2, BG, step=2)
        def _g(g):
            drain_gathers(1)
            drain_writes(0)
            fire_gathers(g, 0)
            fire_writes(g - 1, 1)
            drain_gathers(0)
            drain_writes(1)
            fire_gathers(g + 1, 1)
            fire_writes(g, 0)

        drain_gathers(1)
        drain_writes(0)
        fire_writes(BG - 1, 1)
        drain_writes(1)

    return gather_kernel(table2d, idx4d)


def _mlp_body(emb_ref, num_ref, w1a_ref, w1b_ref, b1_ref, w2_ref, b2_ref,
              bnw_ref, bnb_ref, out_ref):
    x = emb_ref[...]
    h = jnp.dot(x, w1a_ref[...], preferred_element_type=jnp.float32)
    num_n = num_ref[...] * (bnw_ref[...] * _BN_INV) + bnb_ref[...]
    h = h + jnp.dot(num_n, w1b_ref[...], preferred_element_type=jnp.float32)
    h = jnp.maximum(h + b1_ref[...], 0.0)
    out_ref[...] = jnp.dot(h, w2_ref[...], preferred_element_type=jnp.float32) + b2_ref[...]


def _mlp(emb_concat, num, W1, b1, W2, b2, bn_w, bn_b):
    BM = 1024
    W1a = W1[: F * D]
    W1b = W1[F * D :]
    return pl.pallas_call(
        _mlp_body,
        grid=(B // BM,),
        in_specs=[
            pl.BlockSpec((BM, F * D), lambda i: (i, 0)),
            pl.BlockSpec((BM, ND), lambda i: (i, 0)),
            pl.BlockSpec((F * D, H), lambda i: (0, 0)),
            pl.BlockSpec((ND, H), lambda i: (0, 0)),
            pl.BlockSpec((1, H), lambda i: (0, 0)),
            pl.BlockSpec((H, 1), lambda i: (0, 0)),
            pl.BlockSpec((1, 1), lambda i: (0, 0)),
            pl.BlockSpec((1, ND), lambda i: (0, 0)),
            pl.BlockSpec((1, ND), lambda i: (0, 0)),
        ],
        out_specs=pl.BlockSpec((BM, 1), lambda i: (i, 0)),
        out_shape=jax.ShapeDtypeStruct((B, 1), jnp.float32),
    )(emb_concat, num, W1a, W1b, b1.reshape(1, H), W2, b2.reshape(1, 1),
      bn_w.reshape(1, ND), bn_b.reshape(1, ND))


def kernel(cat_idx, num, emb_tables, W1, b1, W2, b2, bn_w, bn_b):
    t_view = jnp.transpose(emb_tables, (0, 2, 1))
    table2d = _tc_transpose(t_view).reshape(F * VP, D)
    v = cat_idx.astype(jnp.int32)
    perm_idx = (v // SCW) * SCW + (v % 32) * 32 + (v % SCW) // 32
    flat_idx = perm_idx + (jnp.arange(F, dtype=jnp.int32) * VP)[None, :]
    idx4d = flat_idx.reshape(NW, BG, GL, F).transpose(0, 1, 3, 2)
    emb_concat = _sc_gather(table2d, idx4d)
    out = _mlp(emb_concat, num, W1, b1, W2, b2, bn_w, bn_b)
    return (out, emb_concat)


# speedup vs baseline: 8.6139x; 2.2475x over previous
import functools

import jax
import jax.numpy as jnp
from jax import lax
from jax.experimental import pallas as pl
from jax.experimental.pallas import tpu as pltpu
from jax.experimental.pallas import tpu_sc as plsc

B = 16384
F = 26
V = 100000
D = 32
ND = 13
H = 64

NC = 2
NS = 16
NW = NC * NS

BPW = B // NW
GL = 64
BG = BPW // GL

GW = 4096
NG = V // GW
VP = (NG + 1) * GW
T0 = NG * GW
T1 = V - 1024

_BN_INV = 1.0 / (1.0 + 1e-5) ** 0.5


def _transpose_group(in_ref, out_ref, base, offs):
    xs = [in_ref[0, :, pl.ds(o, 1024)] for o in offs]
    x4 = jnp.concatenate(xs, axis=0)
    y = jnp.swapaxes(x4, 0, 1)
    out_ref[pl.ds(base, GW * D)] = y.reshape(GW * D)


def _transpose_body(in_ref, out_ref):
    def group(g, carry):
        offs = [g * GW + j * 1024 for j in range(4)]
        _transpose_group(in_ref, out_ref, g * GW * D, offs)
        return carry

    jax.lax.fori_loop(0, NG, group, 0, unroll=2)
    _transpose_group(in_ref, out_ref, NG * GW * D, [T0, T1, T0, T0])


def _tc_transpose(t_view):
    return pl.pallas_call(
        _transpose_body,
        grid=(F,),
        in_specs=[pl.BlockSpec((1, D, V), lambda f: (f, 0, 0))],
        out_specs=pl.BlockSpec((VP * D,), lambda f: (f,)),
        out_shape=jax.ShapeDtypeStruct((F * VP * D,), jnp.float32),
    )(t_view)


def _sc_gather(table2d, idx4d):
    mesh = plsc.VectorSubcoreMesh(core_axis_name="c", subcore_axis_name="s")

    @functools.partial(
        pl.kernel,
        out_type=jax.ShapeDtypeStruct((B, F * D), jnp.float32),
        mesh=mesh,
        scratch_types=[
            pltpu.VMEM((BG, F, GL), jnp.int32),
            pltpu.VMEM((2, F, GL, D), jnp.float32),
            pltpu.SemaphoreType.DMA,
            pltpu.SemaphoreType.DMA,
        ],
        compiler_params=pltpu.CompilerParams(use_tc_tiling_on_sc=False),
    )
    def gather_kernel(table_hbm, idx_hbm, out_hbm, idx_v, buf_v, gsem, wsem):
        wid = lax.axis_index("s") * NC + lax.axis_index("c")
        base = wid * BPW
        pltpu.sync_copy(idx_hbm.at[wid], idx_v)

        def fire_gathers(g, p):
            @pl.loop(0, F)
            def _f(f):
                pltpu.async_copy(
                    table_hbm.at[idx_v.at[g, f]],
                    buf_v.at[p, f],
                    gsem,
                )

        def drain_gathers(p):
            @pl.loop(0, F)
            def _f(f):
                pltpu.make_async_copy(
                    table_hbm.at[idx_v.at[0, 0]], buf_v.at[p, 0], gsem
                ).wait()

        def fire_writes(g, p):
            b0 = base + g * GL

            @pl.loop(0, F)
            def _f(f):
                pltpu.async_copy(
                    buf_v.at[p, f],
                    out_hbm.at[pl.ds(b0, GL), pl.ds(f * D, D)],
                    wsem,
                )

        def drain_writes(p):
            @pl.loop(0, F)
            def _f(f):
                pltpu.make_async_copy(
                    buf_v.at[p, 0],
                    out_hbm.at[pl.ds(base, GL), pl.ds(0, D)],
                    wsem,
                ).wait()

        fire_gathers(0, 0)
        drain_gathers(0)
        fire_gathers(1, 1)
        fire_writes(0, 0)

        @pl.loop(2, BG, step=2)
        def _g(g):
            drain_gathers(1)
            drain_writes(0)
            fire_gathers(g, 0)
            fire_writes(g - 1, 1)
            drain_gathers(0)
            drain_writes(1)
            fire_gathers(g + 1, 1)
            fire_writes(g, 0)

        drain_gathers(1)
        drain_writes(0)
        fire_writes(BG - 1, 1)
        drain_writes(1)

    return gather_kernel(table2d, idx4d)


def _mlp_body(emb_ref, num_ref, w1a_ref, w1b_ref, b1_ref, w2_ref, b2_ref,
              bnw_ref, bnb_ref, out_ref):
    x = emb_ref[...]
    h = jnp.dot(x, w1a_ref[...], preferred_element_type=jnp.float32)
    num_n = num_ref[...] * (bnw_ref[...] * _BN_INV) + bnb_ref[...]
    h = h + jnp.dot(num_n, w1b_ref[...], preferred_element_type=jnp.float32)
    h = jnp.maximum(h + b1_ref[...], 0.0)
    out_ref[...] = jnp.dot(h, w2_ref[...], preferred_element_type=jnp.float32) + b2_ref[...]


def _mlp(emb_concat, num, W1, b1, W2, b2, bn_w, bn_b):
    BM = 1024
    W1a = W1[: F * D]
    W1b = W1[F * D :]
    return pl.pallas_call(
        _mlp_body,
        grid=(B // BM,),
        in_specs=[
            pl.BlockSpec((BM, F * D), lambda i: (i, 0)),
            pl.BlockSpec((BM, ND), lambda i: (i, 0)),
            pl.BlockSpec((F * D, H), lambda i: (0, 0)),
            pl.BlockSpec((ND, H), lambda i: (0, 0)),
            pl.BlockSpec((1, H), lambda i: (0, 0)),
            pl.BlockSpec((H, 1), lambda i: (0, 0)),
            pl.BlockSpec((1, 1), lambda i: (0, 0)),
            pl.BlockSpec((1, ND), lambda i: (0, 0)),
            pl.BlockSpec((1, ND), lambda i: (0, 0)),
        ],
        out_specs=pl.BlockSpec((BM, 1), lambda i: (i, 0)),
        out_shape=jax.ShapeDtypeStruct((B, 1), jnp.float32),
    )(emb_concat, num, W1a, W1b, b1.reshape(1, H), W2, b2.reshape(1, 1),
      bn_w.reshape(1, ND), bn_b.reshape(1, ND))


def kernel(cat_idx, num, emb_tables, W1, b1, W2, b2, bn_w, bn_b):
    t_view = jnp.transpose(emb_tables, (0, 2, 1))
    table2d = _tc_transpose(t_view).reshape(F * VP, D)
    v = cat_idx.astype(jnp.int32)
    perm_idx = jnp.where(
        v < T0,
        (v // GW) * GW + (v % 1024) * 4 + (v % GW) // 1024,
        jnp.where(v < T0 + 1024,
                  T0 + (v - T0) * 4,
                  T0 + (v - T1) * 4 + 1),
    )
    flat_idx = perm_idx + (jnp.arange(F, dtype=jnp.int32) * VP)[None, :]
    idx4d = flat_idx.reshape(NW, BG, GL, F).transpose(0, 1, 3, 2)
    emb_concat = _sc_gather(table2d, idx4d)
    out = _mlp(emb_concat, num, W1, b1, W2, b2, bn_w, bn_b)
    return (out, emb_concat)


# speedup vs baseline: 8.6493x; 1.0041x over previous
import functools

import jax
import jax.numpy as jnp
from jax import lax
from jax.experimental import pallas as pl
from jax.experimental.pallas import tpu as pltpu
from jax.experimental.pallas import tpu_sc as plsc

B = 16384
F = 26
V = 100000
D = 32
ND = 13
H = 64

NC = 2
NS = 16
NW = NC * NS

BPW = B // NW
GL = 64
BG = BPW // GL

GW = 4096
NG = V // GW
VP = (NG + 1) * GW
T0 = NG * GW
T1 = V - 1024

_BN_INV = 1.0 / (1.0 + 1e-5) ** 0.5


def _transpose_group(in_ref, out_ref, base, offs):
    xs = [in_ref[0, :, pl.ds(o, 1024)] for o in offs]
    x4 = jnp.concatenate(xs, axis=0)
    y = jnp.swapaxes(x4, 0, 1)
    out_ref[pl.ds(base, GW * D)] = y.reshape(GW * D)


def _transpose_body(in_ref, out_ref):
    def group(g, carry):
        offs = [g * GW + j * 1024 for j in range(4)]
        _transpose_group(in_ref, out_ref, g * GW * D, offs)
        return carry

    jax.lax.fori_loop(0, NG, group, 0, unroll=4)
    _transpose_group(in_ref, out_ref, NG * GW * D, [T0, T1, T0, T0])


def _tc_transpose(t_view):
    return pl.pallas_call(
        _transpose_body,
        grid=(F,),
        in_specs=[pl.BlockSpec((1, D, V), lambda f: (f, 0, 0))],
        out_specs=pl.BlockSpec((VP * D,), lambda f: (f,)),
        out_shape=jax.ShapeDtypeStruct((F * VP * D,), jnp.float32),
    )(t_view)


def _sc_gather(table2d, idx4d):
    mesh = plsc.VectorSubcoreMesh(core_axis_name="c", subcore_axis_name="s")

    @functools.partial(
        pl.kernel,
        out_type=jax.ShapeDtypeStruct((B, F * D), jnp.float32),
        mesh=mesh,
        scratch_types=[
            pltpu.VMEM((BG, F, GL), jnp.int32),
            pltpu.VMEM((2, F, GL, D), jnp.float32),
            pltpu.SemaphoreType.DMA,
            pltpu.SemaphoreType.DMA,
        ],
        compiler_params=pltpu.CompilerParams(use_tc_tiling_on_sc=False),
    )
    def gather_kernel(table_hbm, idx_hbm, out_hbm, idx_v, buf_v, gsem, wsem):
        wid = lax.axis_index("s") * NC + lax.axis_index("c")
        base = wid * BPW
        pltpu.sync_copy(idx_hbm.at[wid], idx_v)

        def fire_gathers(g, p):
            @pl.loop(0, F)
            def _f(f):
                pltpu.async_copy(
                    table_hbm.at[idx_v.at[g, f]],
                    buf_v.at[p, f],
                    gsem,
                )

        def drain_gathers(p):
            @pl.loop(0, F)
            def _f(f):
                pltpu.make_async_copy(
                    table_hbm.at[idx_v.at[0, 0]], buf_v.at[p, 0], gsem
                ).wait()

        def fire_writes(g, p):
            b0 = base + g * GL

            @pl.loop(0, F)
            def _f(f):
                pltpu.async_copy(
                    buf_v.at[p, f],
                    out_hbm.at[pl.ds(b0, GL), pl.ds(f * D, D)],
                    wsem,
                )

        def drain_writes(p):
            @pl.loop(0, F)
            def _f(f):
                pltpu.make_async_copy(
                    buf_v.at[p, 0],
                    out_hbm.at[pl.ds(base, GL), pl.ds(0, D)],
                    wsem,
                ).wait()

        fire_gathers(0, 0)
        drain_gathers(0)
        fire_gathers(1, 1)
        fire_writes(0, 0)

        @pl.loop(2, BG, step=2)
        def _g(g):
            drain_gathers(1)
            drain_writes(0)
            fire_gathers(g, 0)
            fire_writes(g - 1, 1)
            drain_gathers(0)
            drain_writes(1)
            fire_gathers(g + 1, 1)
            fire_writes(g, 0)

        drain_gathers(1)
        drain_writes(0)
        fire_writes(BG - 1, 1)
        drain_writes(1)

    return gather_kernel(table2d, idx4d)


def _mlp_body(emb_ref, num_ref, w1a_ref, w1b_ref, b1_ref, w2_ref, b2_ref,
              bnw_ref, bnb_ref, out_ref):
    x = emb_ref[...]
    h = jnp.dot(x, w1a_ref[...], preferred_element_type=jnp.float32)
    num_n = num_ref[...] * (bnw_ref[...] * _BN_INV) + bnb_ref[...]
    h = h + jnp.dot(num_n, w1b_ref[...], preferred_element_type=jnp.float32)
    h = jnp.maximum(h + b1_ref[...], 0.0)
    out_ref[...] = jnp.dot(h, w2_ref[...], preferred_element_type=jnp.float32) + b2_ref[...]


def _mlp(emb_concat, num, W1, b1, W2, b2, bn_w, bn_b):
    BM = 2048
    W1a = W1[: F * D]
    W1b = W1[F * D :]
    return pl.pallas_call(
        _mlp_body,
        grid=(B // BM,),
        in_specs=[
            pl.BlockSpec((BM, F * D), lambda i: (i, 0)),
            pl.BlockSpec((BM, ND), lambda i: (i, 0)),
            pl.BlockSpec((F * D, H), lambda i: (0, 0)),
            pl.BlockSpec((ND, H), lambda i: (0, 0)),
            pl.BlockSpec((1, H), lambda i: (0, 0)),
            pl.BlockSpec((H, 1), lambda i: (0, 0)),
            pl.BlockSpec((1, 1), lambda i: (0, 0)),
            pl.BlockSpec((1, ND), lambda i: (0, 0)),
            pl.BlockSpec((1, ND), lambda i: (0, 0)),
        ],
        out_specs=pl.BlockSpec((BM, 1), lambda i: (i, 0)),
        out_shape=jax.ShapeDtypeStruct((B, 1), jnp.float32),
    )(emb_concat, num, W1a, W1b, b1.reshape(1, H), W2, b2.reshape(1, 1),
      bn_w.reshape(1, ND), bn_b.reshape(1, ND))


def kernel(cat_idx, num, emb_tables, W1, b1, W2, b2, bn_w, bn_b):
    t_view = jnp.transpose(emb_tables, (0, 2, 1))
    table2d = _tc_transpose(t_view).reshape(F * VP, D)
    v = cat_idx.astype(jnp.int32)
    perm_idx = jnp.where(
        v < T0,
        (v // GW) * GW + (v % 1024) * 4 + (v % GW) // 1024,
        jnp.where(v < T0 + 1024,
                  T0 + (v - T0) * 4,
                  T0 + (v - T1) * 4 + 1),
    )
    flat_idx = perm_idx + (jnp.arange(F, dtype=jnp.int32) * VP)[None, :]
    idx4d = flat_idx.reshape(NW, BG, GL, F).transpose(0, 1, 3, 2)
    emb_concat = _sc_gather(table2d, idx4d)
    out = _mlp(emb_concat, num, W1, b1, W2, b2, bn_w, bn_b)
    return (out, emb_concat)
